# baseline (speedup 1.0000x reference)
import jax
import jax.numpy as jnp
from jax.experimental import pallas as pl
from jax.experimental.pallas import tpu as pltpu

D = 512
H = 8
DK = D // H
INV_SQRT_DK = 1.0 / (DK ** 0.5)


def _pool_kernel(r_ref, k1_ref, k2_ref, mask_ref, wq_ref, bq_ref, wk_ref,
                 bk_ref, wv_ref, bv_ref, wo_ref, bo_ref, attn_ref,
                 pooled_ref):
    f32 = jnp.float32
    q = jax.lax.dot_general(
        r_ref[0], wq_ref[...], (((1,), (1,)), ((), ())),
        preferred_element_type=f32) + bq_ref[...]
    h_ids = jax.lax.broadcasted_iota(jnp.int32, (H, D), 0)
    d_ids = jax.lax.broadcasted_iota(jnp.int32, (H, D), 1)
    hm = (d_ids // DK) == h_ids
    m8 = jnp.where(hm, jnp.broadcast_to(q, (H, D)), f32(0.0))
    a_s = jax.lax.dot_general(
        m8, wk_ref[...], (((1,), (0,)), ((), ())),
        preferred_element_type=f32) * f32(INV_SQRT_DK)
    c_s = jnp.sum(m8 * bk_ref[...], axis=1, keepdims=True) * f32(INV_SQRT_DK)

    k1 = k1_ref[0]
    k2 = k2_ref[0]
    s1 = jax.lax.dot_general(
        a_s, k1, (((1,), (1,)), ((), ())),
        preferred_element_type=f32)
    s2 = jax.lax.dot_general(
        a_s, k2, (((1,), (1,)), ((), ())),
        preferred_element_type=f32)
    s = jnp.concatenate([s1, s2], axis=1) + c_s
    mrow = mask_ref[0]
    s = jnp.where(mrow != f32(0.0), s, f32(-1e9))

    m = jnp.max(s, axis=1, keepdims=True)
    p = jnp.exp(s - m)
    l = jnp.sum(p, axis=1, keepdims=True)
    rl = f32(1.0) / l
    attn_ref[0] = p * rl

    hr = p.shape[1] // 2
    pn = (jax.lax.dot_general(
              p[:, :hr], k1, (((1,), (0,)), ((), ())),
              preferred_element_type=f32)
          + jax.lax.dot_general(
              p[:, hr:], k2, (((1,), (0,)), ((), ())),
              preferred_element_type=f32)) * rl
    g = jax.lax.dot_general(
        pn, wv_ref[...], (((1,), (1,)), ((), ())),
        preferred_element_type=f32)
    pooled = jnp.sum(jnp.where(hm, g, f32(0.0)), axis=0, keepdims=True)
    pooled = pooled + bv_ref[...]
    out = jax.lax.dot_general(
        pooled, wo_ref[...], (((1,), (1,)), ((), ())),
        preferred_element_type=f32) + bo_ref[...]
    pooled_ref[0] = out


def kernel(r, K, mask, Wq, bq, Wk, bk, Wv, bv, Wo, bo):
    B, R, d = K.shape
    r3 = r.reshape(B, 1, d)
    mask3 = mask.astype(jnp.float32).reshape(B, 1, R)
    b2 = [b.reshape(1, d) for b in (bq, bk, bv, bo)]

    wspec = pl.BlockSpec((d, d), lambda b: (0, 0))
    bspec = pl.BlockSpec((1, d), lambda b: (0, 0))
    attn, pooled3 = pl.pallas_call(
        _pool_kernel,
        grid=(B,),
        in_specs=[
            pl.BlockSpec((1, 1, d), lambda b: (b, 0, 0)),
            pl.BlockSpec((1, R // 2, d), lambda b: (b, 0, 0)),
            pl.BlockSpec((1, R // 2, d), lambda b: (b, 1, 0)),
            pl.BlockSpec((1, 1, R), lambda b: (b, 0, 0)),
            wspec, bspec,
            wspec, bspec,
            wspec, bspec,
            wspec, bspec,
        ],
        out_specs=[
            pl.BlockSpec((1, H, R), lambda b: (b, 0, 0)),
            pl.BlockSpec((1, 1, d), lambda b: (b, 0, 0)),
        ],
        out_shape=[
            jax.ShapeDtypeStruct((B, H, R), jnp.float32),
            jax.ShapeDtypeStruct((B, 1, d), jnp.float32),
        ],
        compiler_params=pltpu.CompilerParams(
            dimension_semantics=("parallel",),
            vmem_limit_bytes=50 * 1024 * 1024,
        ),
        name="cross_attention_pool",
    )(r3, K, K, mask3, Wq, b2[0], Wk, b2[1], Wv, b2[2], Wo, b2[3])
    return (pooled3.reshape(B, d), attn)


# speedup vs baseline: 1.0285x; 1.0285x over previous
import jax
import jax.numpy as jnp
from jax.experimental import pallas as pl
from jax.experimental.pallas import tpu as pltpu

D = 512
H = 8
DK = D // H
INV_SQRT_DK = 1.0 / (DK ** 0.5)


def _pool_kernel(r_ref, k_ref, mask_ref, wq_ref, bq_ref, wk_ref,
                 wv_ref, bv_ref, wo_ref, bo_ref, attn_ref, pooled_ref,
                 a_scr):
    f32 = jnp.float32
    b = pl.program_id(0)
    nb = pl.num_programs(0)

    @pl.when(b == 0)
    def _precompute_a():
        q_all = jax.lax.dot_general(
            r_ref[...], wq_ref[...], (((1,), (1,)), ((), ())),
            preferred_element_type=f32) + bq_ref[...]
        rows = jax.lax.broadcasted_iota(jnp.int32, (nb * H, nb), 0)
        cols = jax.lax.broadcasted_iota(jnp.int32, (nb * H, nb), 1)
        expand = jnp.where(rows // H == cols, f32(1.0), f32(0.0))
        q_rep = jax.lax.dot_general(
            expand, q_all, (((1,), (0,)), ((), ())),
            preferred_element_type=f32)
        h_ids = jax.lax.broadcasted_iota(jnp.int32, (nb * H, D), 0) % H
        d_ids = jax.lax.broadcasted_iota(jnp.int32, (nb * H, D), 1)
        m_all = jnp.where(d_ids // DK == h_ids, q_rep, f32(0.0))
        a_all = jax.lax.dot_general(
            m_all, wk_ref[...], (((1,), (0,)), ((), ())),
            preferred_element_type=f32) * f32(INV_SQRT_DK)
        a_scr[...] = a_all.reshape(nb, H, D)

    a_s = a_scr[pl.ds(b, 1)].reshape(H, D)
    kb = k_ref[0]
    s = jax.lax.dot_general(
        a_s, kb, (((1,), (1,)), ((), ())),
        preferred_element_type=f32)
    mrow = mask_ref[0]
    s = jnp.where(mrow != f32(0.0), s, f32(-1e9))

    m = jnp.max(s, axis=1, keepdims=True)
    p = jnp.exp(s - m)
    l = jnp.sum(p, axis=1, keepdims=True)
    rl = f32(1.0) / l
    attn_ref[0] = p * rl

    pn = jax.lax.dot_general(
        p, kb, (((1,), (0,)), ((), ())),
        preferred_element_type=f32) * rl
    g = jax.lax.dot_general(
        pn, wv_ref[...], (((1,), (1,)), ((), ())),
        preferred_element_type=f32)
    h_ids2 = jax.lax.broadcasted_iota(jnp.int32, (H, D), 0)
    d_ids2 = jax.lax.broadcasted_iota(jnp.int32, (H, D), 1)
    hm = (d_ids2 // DK) == h_ids2
    pooled = jnp.sum(jnp.where(hm, g, f32(0.0)), axis=0, keepdims=True)
    pooled = pooled + bv_ref[...]
    out = jax.lax.dot_general(
        pooled, wo_ref[...], (((1,), (1,)), ((), ())),
        preferred_element_type=f32) + bo_ref[...]
    pooled_ref[0] = out


def kernel(r, K, mask, Wq, bq, Wk, bk, Wv, bv, Wo, bo):
    B, R, d = K.shape
    mask3 = mask.astype(jnp.float32).reshape(B, 1, R)
    bq2, bv2, bo2 = (b.reshape(1, d) for b in (bq, bv, bo))

    wspec = pl.BlockSpec((d, d), lambda b: (0, 0))
    bspec = pl.BlockSpec((1, d), lambda b: (0, 0))
    attn, pooled3 = pl.pallas_call(
        _pool_kernel,
        grid=(B,),
        in_specs=[
            pl.BlockSpec((B, d), lambda b: (0, 0)),
            pl.BlockSpec((1, R, d), lambda b: (b, 0, 0)),
            pl.BlockSpec((1, 1, R), lambda b: (b, 0, 0)),
            wspec, bspec,
            wspec,
            wspec, bspec,
            wspec, bspec,
        ],
        out_specs=[
            pl.BlockSpec((1, H, R), lambda b: (b, 0, 0)),
            pl.BlockSpec((1, 1, d), lambda b: (b, 0, 0)),
        ],
        out_shape=[
            jax.ShapeDtypeStruct((B, H, R), jnp.float32),
            jax.ShapeDtypeStruct((B, 1, d), jnp.float32),
        ],
        scratch_shapes=[pltpu.VMEM((B, H, d), jnp.float32)],
        compiler_params=pltpu.CompilerParams(
            dimension_semantics=("arbitrary",),
            vmem_limit_bytes=50 * 1024 * 1024,
        ),
        name="cross_attention_pool",
    )(r, K, mask3, Wq, bq2, Wk, Wv, bv2, Wo, bo2)
    return (pooled3.reshape(B, d), attn)


# speedup vs baseline: 1.0373x; 1.0086x over previous
import jax
import jax.numpy as jnp
from jax.experimental import pallas as pl
from jax.experimental.pallas import tpu as pltpu

D = 512
H = 8
DK = D // H
INV_SQRT_DK = 1.0 / (DK ** 0.5)


def _pool_kernel(r_ref, k_ref, mask_ref, wq_ref, bq_ref, wk_ref,
                 wv_ref, bv_ref, wo_ref, bo_ref, attn_ref, pooled_ref,
                 a_scr, pn_scr):
    f32 = jnp.float32
    b = pl.program_id(0)
    nb = pl.num_programs(0)

    @pl.when(b == 0)
    def _precompute_a():
        q_all = jax.lax.dot_general(
            r_ref[...], wq_ref[...], (((1,), (1,)), ((), ())),
            preferred_element_type=f32) + bq_ref[...]
        rows = jax.lax.broadcasted_iota(jnp.int32, (nb * H, nb), 0)
        cols = jax.lax.broadcasted_iota(jnp.int32, (nb * H, nb), 1)
        expand = jnp.where(rows // H == cols, f32(1.0), f32(0.0))
        q_rep = jax.lax.dot_general(
            expand, q_all, (((1,), (0,)), ((), ())),
            preferred_element_type=f32)
        h_ids = jax.lax.broadcasted_iota(jnp.int32, (nb * H, D), 0) % H
        d_ids = jax.lax.broadcasted_iota(jnp.int32, (nb * H, D), 1)
        m_all = jnp.where(d_ids // DK == h_ids, q_rep, f32(0.0))
        a_all = jax.lax.dot_general(
            m_all, wk_ref[...], (((1,), (0,)), ((), ())),
            preferred_element_type=f32) * f32(INV_SQRT_DK)
        a_scr[...] = a_all.reshape(nb, H, D)

    a_s = a_scr[pl.ds(b, 1)].reshape(H, D)
    kb = k_ref[0]
    s = jax.lax.dot_general(
        a_s, kb, (((1,), (1,)), ((), ())),
        preferred_element_type=f32)
    mrow = mask_ref[0]
    s = jnp.where(mrow != f32(0.0), s, f32(-1e9))

    m = jnp.max(s, axis=1, keepdims=True)
    p = jnp.exp(s - m)
    l = jnp.sum(p, axis=1, keepdims=True)
    rl = f32(1.0) / l
    attn_ref[0] = p * rl

    pn = jax.lax.dot_general(
        p, kb, (((1,), (0,)), ((), ())),
        preferred_element_type=f32) * rl
    pn_scr[pl.ds(b, 1)] = pn.reshape(1, H, D)

    @pl.when(b == nb - 1)
    def _project_all():
        pn_all = pn_scr[...].reshape(nb * H, D)
        g_all = jax.lax.dot_general(
            pn_all, wv_ref[...], (((1,), (1,)), ((), ())),
            preferred_element_type=f32)
        h_ids2 = jax.lax.broadcasted_iota(jnp.int32, (nb, H, D), 1)
        d_ids2 = jax.lax.broadcasted_iota(jnp.int32, (nb, H, D), 2)
        hm = (d_ids2 // DK) == h_ids2
        g3 = jnp.where(hm, g_all.reshape(nb, H, D), f32(0.0))
        pooled = jnp.sum(g3, axis=1) + bv_ref[...]
        out = jax.lax.dot_general(
            pooled, wo_ref[...], (((1,), (1,)), ((), ())),
            preferred_element_type=f32) + bo_ref[...]
        pooled_ref[...] = out.reshape(nb, 1, D)


def kernel(r, K, mask, Wq, bq, Wk, bk, Wv, bv, Wo, bo):
    B, R, d = K.shape
    mask3 = mask.astype(jnp.float32).reshape(B, 1, R)
    bq2, bv2, bo2 = (b.reshape(1, d) for b in (bq, bv, bo))

    wspec = pl.BlockSpec((d, d), lambda b: (0, 0))
    bspec = pl.BlockSpec((1, d), lambda b: (0, 0))
    attn, pooled3 = pl.pallas_call(
        _pool_kernel,
        grid=(B,),
        in_specs=[
            pl.BlockSpec((B, d), lambda b: (0, 0)),
            pl.BlockSpec((1, R, d), lambda b: (b, 0, 0)),
            pl.BlockSpec((1, 1, R), lambda b: (b, 0, 0)),
            wspec, bspec,
            wspec,
            wspec, bspec,
            wspec, bspec,
        ],
        out_specs=[
            pl.BlockSpec((1, H, R), lambda b: (b, 0, 0)),
            pl.BlockSpec((B, 1, d), lambda b: (0, 0, 0)),
        ],
        out_shape=[
            jax.ShapeDtypeStruct((B, H, R), jnp.float32),
            jax.ShapeDtypeStruct((B, 1, d), jnp.float32),
        ],
        scratch_shapes=[pltpu.VMEM((B, H, d), jnp.float32),
                        pltpu.VMEM((B, H, d), jnp.float32)],
        compiler_params=pltpu.CompilerParams(
            dimension_semantics=("arbitrary",),
            vmem_limit_bytes=50 * 1024 * 1024,
        ),
        name="cross_attention_pool",
    )(r, K, mask3, Wq, bq2, Wk, Wv, bv2, Wo, bo2)
    return (pooled3.reshape(B, d), attn)
